# Initial kernel scaffold; baseline (speedup 1.0000x reference)
import jax
import jax.numpy as jnp
from jax.experimental import pallas as pl


def kernel(x, edge_index, W1, b1, W2, b2):
    raise NotImplementedError("write your pallas kernel here")


# speedup vs baseline: 17.4489x; 17.4489x over previous
import functools

import jax
import jax.numpy as jnp
from jax import lax
from jax.experimental import pallas as pl
from jax.experimental.pallas import tpu as pltpu
from jax.experimental.pallas import tpu_sc as plsc

N = 10000
NPAD = 10240
E = 320000
NC = 2
NS = 16
DH = 64
CHUNK = 80
NCHUNKS = E // NS // CHUNK
ROWS_PER_TILE = NPAD // NS
WCHUNK = 128
DEG_NCHUNKS = (E // (NC * NS)) // CHUNK

RT = 1024
GRID = NPAD // RT

_SC_PARAMS = pltpu.CompilerParams(use_tc_tiling_on_sc=False)


def _sc_mesh():
    return plsc.VectorSubcoreMesh(
        core_axis_name="c", subcore_axis_name="s", num_cores=NC, num_subcores=NS
    )


@functools.cache
def _build_deg_kernel():
    @functools.partial(
        pl.kernel,
        out_type=jax.ShapeDtypeStruct((NC, NPAD, 16), jnp.float32),
        mesh=_sc_mesh(),
        compiler_params=_SC_PARAMS,
        scratch_types=[
            pltpu.VMEM((DEG_NCHUNKS, CHUNK), jnp.int32),
            pltpu.VMEM((CHUNK, 16), jnp.float32),
            pltpu.VMEM((ROWS_PER_TILE, 16), jnp.float32),
            pltpu.VMEM_SHARED((NPAD, 16), jnp.float32),
        ],
    )
    def _deg_kernel(dstr32, out, didx, ones_v, zeros_v, acc):
        cid = lax.axis_index("c")
        sid = lax.axis_index("s")
        wid = cid * NS + sid
        stripe = pl.ds(sid * ROWS_PER_TILE, ROWS_PER_TILE)
        pltpu.sync_copy(dstr32.at[wid], didx)

        def fill(i, carry):
            ones_v[i] = jnp.ones((16,), jnp.float32)
            return carry

        lax.fori_loop(0, CHUNK, fill, 0)

        def zfill(i, carry):
            zeros_v[i] = jnp.zeros((16,), jnp.float32)
            return carry

        lax.fori_loop(0, ROWS_PER_TILE, zfill, 0)
        pltpu.sync_copy(zeros_v, acc.at[stripe])
        plsc.subcore_barrier()

        def body(j, carry):
            pltpu.sync_copy(ones_v, acc.at[didx.at[j]], add=True)
            return carry

        lax.fori_loop(0, DEG_NCHUNKS, body, 0)
        plsc.subcore_barrier()
        pltpu.sync_copy(acc.at[stripe], zeros_v)
        pltpu.sync_copy(zeros_v, out.at[cid, stripe])

    return _deg_kernel


@functools.cache
def _build_spmm():
    @functools.partial(
        pl.kernel,
        out_type=jax.ShapeDtypeStruct((NC, NPAD, DH), jnp.float32),
        mesh=_sc_mesh(),
        compiler_params=_SC_PARAMS,
        scratch_types=[
            pltpu.VMEM((NCHUNKS, CHUNK), jnp.int32),
            pltpu.VMEM((NCHUNKS, CHUNK), jnp.int32),
            pltpu.VMEM((2, CHUNK, DH), jnp.float32),
            pltpu.VMEM((WCHUNK, DH), jnp.float32),
            pltpu.VMEM_SHARED((NPAD, DH), jnp.float32),
            pltpu.SemaphoreType.DMA((2,)),
        ],
    )
    def _spmm(srcr, dstr, hs_flat, out, sidx, didx, rows, wbuf, acc, sem):
        cid = lax.axis_index("c")
        sid = lax.axis_index("s")
        base = sid * ROWS_PER_TILE
        pltpu.sync_copy(srcr.at[cid, sid], sidx)
        pltpu.sync_copy(dstr.at[cid, sid], didx)
        for w in range(ROWS_PER_TILE // WCHUNK):
            pltpu.sync_copy(
                hs_flat.at[pl.ds(cid * NPAD + base + w * WCHUNK, WCHUNK)], wbuf
            )
            pltpu.sync_copy(wbuf, acc.at[pl.ds(base + w * WCHUNK, WCHUNK)])
        plsc.subcore_barrier()
        pltpu.async_copy(hs_flat.at[sidx.at[0]], rows.at[0], sem.at[0])

        def outer(t, carry):
            for b in range(2):
                j = 2 * t + b

                @pl.when(j + 1 < NCHUNKS)
                def _start_next():
                    pltpu.async_copy(
                        hs_flat.at[sidx.at[j + 1]], rows.at[1 - b], sem.at[1 - b]
                    )

                pltpu.make_async_copy(
                    hs_flat.at[sidx.at[j]], rows.at[b], sem.at[b]
                ).wait()
                pltpu.sync_copy(rows.at[b], acc.at[didx.at[j]], add=True)
            return carry

        lax.fori_loop(0, NCHUNKS // 2, outer, 0)
        plsc.subcore_barrier()
        for w in range(ROWS_PER_TILE // WCHUNK):
            pltpu.sync_copy(acc.at[pl.ds(base + w * WCHUNK, WCHUNK)], wbuf)
            pltpu.sync_copy(
                wbuf, out.at[cid, pl.ds(base + w * WCHUNK, WCHUNK)]
            )

    return _spmm


def _dinv_of(dr_ref):
    deg = dr_ref[0, :, 0:1] + dr_ref[1, :, 0:1] + 1.0
    return lax.rsqrt(deg)


def _mm1_body(x_ref, w_ref, dr_ref, o_ref):
    dinv = _dinv_of(dr_ref)
    hs = jnp.dot(x_ref[...], w_ref[...], preferred_element_type=jnp.float32) * dinv
    for k in range(4):
        o_ref[k] = hs[:, k * DH:(k + 1) * DH]


def _mm1(x, w1, deg_raw):
    return pl.pallas_call(
        _mm1_body,
        grid=(GRID,),
        in_specs=[
            pl.BlockSpec((RT, 128), lambda i: (i, 0)),
            pl.BlockSpec((128, 256), lambda i: (0, 0)),
            pl.BlockSpec((2, RT, 16), lambda i: (0, i, 0)),
        ],
        out_specs=pl.BlockSpec((4, RT, DH), lambda i: (0, i, 0)),
        out_shape=jax.ShapeDtypeStruct((4, NPAD, DH), jnp.float32),
    )(x, w1, deg_raw)


def _mm2_body(agg_ref, dr_ref, b1_ref, w2_ref, o_ref):
    dinv = _dinv_of(dr_ref)
    h1a = jnp.maximum(
        jnp.concatenate([agg_ref[0], agg_ref[1]], axis=1) * dinv + b1_ref[0], 0.0
    )
    h1b = jnp.maximum(
        jnp.concatenate([agg_ref[2], agg_ref[3]], axis=1) * dinv + b1_ref[1], 0.0
    )
    h2 = jnp.dot(h1a, w2_ref[0], preferred_element_type=jnp.float32) + jnp.dot(
        h1b, w2_ref[1], preferred_element_type=jnp.float32
    )
    hs2 = h2 * dinv
    o_ref[0] = hs2[:, :DH]
    o_ref[1] = hs2[:, DH:]


def _mm2(agg1, deg_raw, b1_2, w2_2):
    return pl.pallas_call(
        _mm2_body,
        grid=(GRID,),
        in_specs=[
            pl.BlockSpec((4, RT, DH), lambda i: (0, i, 0)),
            pl.BlockSpec((2, RT, 16), lambda i: (0, i, 0)),
            pl.BlockSpec((2, 128), lambda i: (0, 0)),
            pl.BlockSpec((2, 128, 128), lambda i: (0, 0, 0)),
        ],
        out_specs=pl.BlockSpec((2, RT, DH), lambda i: (0, i, 0)),
        out_shape=jax.ShapeDtypeStruct((2, NPAD, DH), jnp.float32),
    )(agg1, deg_raw, b1_2, w2_2)


def _epi_body(agg_ref, dr_ref, b2_ref, o_ref):
    dinv = _dinv_of(dr_ref)
    h = jnp.concatenate([agg_ref[0], agg_ref[1]], axis=1)
    o_ref[...] = jnp.maximum(h * dinv + b2_ref[...], 0.0)


def _epi(agg2, deg_raw, b2_2):
    return pl.pallas_call(
        _epi_body,
        grid=(GRID,),
        in_specs=[
            pl.BlockSpec((2, RT, DH), lambda i: (0, i, 0)),
            pl.BlockSpec((2, RT, 16), lambda i: (0, i, 0)),
            pl.BlockSpec((1, 128), lambda i: (0, 0)),
        ],
        out_specs=pl.BlockSpec((RT, 128), lambda i: (i, 0)),
        out_shape=jax.ShapeDtypeStruct((NPAD, 128), jnp.float32),
    )(agg2, deg_raw, b2_2)


def kernel(x, edge_index, W1, b1, W2, b2):
    src = edge_index[0]
    dst = edge_index[1]
    srcr = jnp.stack([src, src + NPAD]).reshape(NC, NS, NCHUNKS, CHUNK)
    dstr = jnp.stack([dst, dst]).reshape(NC, NS, NCHUNKS, CHUNK)
    dstr32 = dst.reshape(NC * NS, DEG_NCHUNKS, CHUNK)
    xp = jnp.pad(x, ((0, NPAD - N), (0, 0)))

    deg_raw = _build_deg_kernel()(dstr32)
    hs1 = _mm1(xp, W1, deg_raw).reshape(2, 2 * NPAD, DH)
    agg1a = _build_spmm()(srcr, dstr, hs1[0])
    agg1b = _build_spmm()(srcr, dstr, hs1[1])
    agg1 = jnp.concatenate([agg1a, agg1b], axis=0)
    hs2 = _mm2(agg1, deg_raw, b1.reshape(2, 128), W2.reshape(2, 128, 128))
    agg2 = _build_spmm()(srcr, dstr, hs2.reshape(2 * NPAD, DH))
    return _epi(agg2, deg_raw, b2.reshape(1, 128))[:N]


# speedup vs baseline: 26.3022x; 1.5074x over previous
import functools

import jax
import jax.numpy as jnp
from jax import lax
from jax.experimental import pallas as pl
from jax.experimental.pallas import tpu as pltpu
from jax.experimental.pallas import tpu_sc as plsc

N = 10000
NPAD = 10240
E = 320000
NC = 2
NS = 16
DH = 64
CHUNK = 125
NBUF = 4
NCHUNKS = E // NS // CHUNK
ROWS_PER_TILE = NPAD // NS
WCHUNK = 128
DEG_NCHUNKS = (E // (NC * NS)) // CHUNK

RT = 2048
GRID = NPAD // RT

_SC_PARAMS = pltpu.CompilerParams(use_tc_tiling_on_sc=False)


def _sc_mesh():
    return plsc.VectorSubcoreMesh(
        core_axis_name="c", subcore_axis_name="s", num_cores=NC, num_subcores=NS
    )


@functools.cache
def _build_deg_kernel():
    @functools.partial(
        pl.kernel,
        out_type=jax.ShapeDtypeStruct((NC, NPAD, 16), jnp.float32),
        mesh=_sc_mesh(),
        compiler_params=_SC_PARAMS,
        scratch_types=[
            pltpu.VMEM((DEG_NCHUNKS, CHUNK), jnp.int32),
            pltpu.VMEM((CHUNK, 16), jnp.float32),
            pltpu.VMEM((ROWS_PER_TILE, 16), jnp.float32),
            pltpu.VMEM_SHARED((NPAD, 16), jnp.float32),
        ],
    )
    def _deg_kernel(dstr32, out, didx, ones_v, zeros_v, acc):
        cid = lax.axis_index("c")
        sid = lax.axis_index("s")
        wid = cid * NS + sid
        stripe = pl.ds(sid * ROWS_PER_TILE, ROWS_PER_TILE)
        pltpu.sync_copy(dstr32.at[wid], didx)

        def fill(i, carry):
            ones_v[i] = jnp.ones((16,), jnp.float32)
            return carry

        lax.fori_loop(0, CHUNK, fill, 0)

        def zfill(i, carry):
            zeros_v[i] = jnp.zeros((16,), jnp.float32)
            return carry

        lax.fori_loop(0, ROWS_PER_TILE, zfill, 0)
        pltpu.sync_copy(zeros_v, acc.at[stripe])
        plsc.subcore_barrier()

        def body(j, carry):
            pltpu.sync_copy(ones_v, acc.at[didx.at[j]], add=True)
            return carry

        lax.fori_loop(0, DEG_NCHUNKS, body, 0)
        plsc.subcore_barrier()
        pltpu.sync_copy(acc.at[stripe], zeros_v)
        pltpu.sync_copy(zeros_v, out.at[cid, stripe])

    return _deg_kernel


@functools.cache
def _build_spmm(npass):
    @functools.partial(
        pl.kernel,
        out_type=jax.ShapeDtypeStruct((2 * npass, NPAD, DH), jnp.float32),
        mesh=_sc_mesh(),
        compiler_params=_SC_PARAMS,
        scratch_types=[
            pltpu.VMEM((npass, NCHUNKS, CHUNK), jnp.int32),
            pltpu.VMEM((NCHUNKS, CHUNK), jnp.int32),
            pltpu.VMEM((NBUF, CHUNK, DH), jnp.float32),
            pltpu.VMEM((2, WCHUNK, DH), jnp.float32),
            pltpu.VMEM_SHARED((NPAD, DH), jnp.float32),
            pltpu.SemaphoreType.DMA((NBUF,)),
            pltpu.SemaphoreType.DMA((2,)),
        ],
    )
    def _spmm(srcr, dstr, hs_flat, out, sidx, didx, rows, wbuf, acc, sem, wsem):
        cid = lax.axis_index("c")
        sid = lax.axis_index("s")
        base = sid * ROWS_PER_TILE
        stripe = pl.ds(base, ROWS_PER_TILE)
        for p in range(npass):
            pltpu.sync_copy(srcr.at[p, cid, sid], sidx.at[p])
        pltpu.sync_copy(dstr.at[sid], didx)

        def one_pass(p, pcarry):
            k = 2 * p + cid
            nw = ROWS_PER_TILE // WCHUNK
            pltpu.async_copy(
                hs_flat.at[pl.ds(k * NPAD + base, WCHUNK)], wbuf.at[0], wsem.at[0]
            )
            for w in range(nw):
                if w + 1 < nw:
                    pltpu.async_copy(
                        hs_flat.at[pl.ds(k * NPAD + base + (w + 1) * WCHUNK, WCHUNK)],
                        wbuf.at[(w + 1) % 2],
                        wsem.at[(w + 1) % 2],
                    )
                pltpu.make_async_copy(
                    hs_flat.at[pl.ds(k * NPAD + base + w * WCHUNK, WCHUNK)],
                    wbuf.at[w % 2],
                    wsem.at[w % 2],
                ).wait()
                pltpu.sync_copy(
                    wbuf.at[w % 2], acc.at[pl.ds(base + w * WCHUNK, WCHUNK)]
                )
            plsc.subcore_barrier()
            for q in range(NBUF - 1):
                pltpu.async_copy(hs_flat.at[sidx.at[p, q]], rows.at[q], sem.at[q])

            def outer(t, carry):
                for b in range(NBUF):
                    j = NBUF * t + b
                    nxt = (b + NBUF - 1) % NBUF

                    @pl.when(j + NBUF - 1 < NCHUNKS)
                    def _start_next():
                        pltpu.async_copy(
                            hs_flat.at[sidx.at[p, j + NBUF - 1]],
                            rows.at[nxt],
                            sem.at[nxt],
                        )

                    pltpu.make_async_copy(
                        hs_flat.at[sidx.at[p, j]], rows.at[b], sem.at[b]
                    ).wait()
                    pltpu.sync_copy(rows.at[b], acc.at[didx.at[j]], add=True)
                return carry

            lax.fori_loop(0, NCHUNKS // NBUF, outer, 0)
            plsc.subcore_barrier()
            for w in range(nw):
                if w >= 2:
                    pltpu.make_async_copy(
                        wbuf.at[w % 2],
                        out.at[k, pl.ds(base + (w - 2) * WCHUNK, WCHUNK)],
                        wsem.at[w % 2],
                    ).wait()
                pltpu.sync_copy(
                    acc.at[pl.ds(base + w * WCHUNK, WCHUNK)], wbuf.at[w % 2]
                )
                pltpu.async_copy(
                    wbuf.at[w % 2],
                    out.at[k, pl.ds(base + w * WCHUNK, WCHUNK)],
                    wsem.at[w % 2],
                )
            for w in range(nw - 2, nw):
                pltpu.make_async_copy(
                    wbuf.at[w % 2],
                    out.at[k, pl.ds(base + w * WCHUNK, WCHUNK)],
                    wsem.at[w % 2],
                ).wait()
            plsc.subcore_barrier()
            return pcarry

        lax.fori_loop(0, npass, one_pass, 0)

    return _spmm


def _dinv_of(dr_ref):
    deg = dr_ref[0, :, 0:1] + dr_ref[1, :, 0:1] + 1.0
    return lax.rsqrt(deg)


def _mm1_body(x_ref, w_ref, dr_ref, o_ref):
    dinv = _dinv_of(dr_ref)
    hs = jnp.dot(x_ref[...], w_ref[...], preferred_element_type=jnp.float32) * dinv
    for k in range(4):
        o_ref[k] = hs[:, k * DH:(k + 1) * DH]


def _mm1(x, w1, deg_raw):
    return pl.pallas_call(
        _mm1_body,
        grid=(GRID,),
        in_specs=[
            pl.BlockSpec((RT, 128), lambda i: (i, 0)),
            pl.BlockSpec((128, 256), lambda i: (0, 0)),
            pl.BlockSpec((2, RT, 16), lambda i: (0, i, 0)),
        ],
        out_specs=pl.BlockSpec((4, RT, DH), lambda i: (0, i, 0)),
        out_shape=jax.ShapeDtypeStruct((4, NPAD, DH), jnp.float32),
    )(x, w1, deg_raw)


def _mm2_body(agga_ref, aggb_ref, dr_ref, b1_ref, w2_ref, o_ref):
    dinv = _dinv_of(dr_ref)
    h1a = jnp.maximum(
        jnp.concatenate([agga_ref[0], agga_ref[1]], axis=1) * dinv + b1_ref[0], 0.0
    )
    h1b = jnp.maximum(
        jnp.concatenate([aggb_ref[0], aggb_ref[1]], axis=1) * dinv + b1_ref[1], 0.0
    )
    h2 = jnp.dot(h1a, w2_ref[0], preferred_element_type=jnp.float32) + jnp.dot(
        h1b, w2_ref[1], preferred_element_type=jnp.float32
    )
    hs2 = h2 * dinv
    o_ref[0] = hs2[:, :DH]
    o_ref[1] = hs2[:, DH:]


def _mm2(agg1a, agg1b, deg_raw, b1_2, w2_2):
    return pl.pallas_call(
        _mm2_body,
        grid=(GRID,),
        in_specs=[
            pl.BlockSpec((2, RT, DH), lambda i: (0, i, 0)),
            pl.BlockSpec((2, RT, DH), lambda i: (0, i, 0)),
            pl.BlockSpec((2, RT, 16), lambda i: (0, i, 0)),
            pl.BlockSpec((2, 128), lambda i: (0, 0)),
            pl.BlockSpec((2, 128, 128), lambda i: (0, 0, 0)),
        ],
        out_specs=pl.BlockSpec((2, RT, DH), lambda i: (0, i, 0)),
        out_shape=jax.ShapeDtypeStruct((2, NPAD, DH), jnp.float32),
    )(agg1a, agg1b, deg_raw, b1_2, w2_2)


def _epi_body(agg_ref, dr_ref, b2_ref, o_ref):
    dinv = _dinv_of(dr_ref)
    h = jnp.concatenate([agg_ref[0], agg_ref[1]], axis=1)
    o_ref[...] = jnp.maximum(h * dinv + b2_ref[...], 0.0)


def _epi(agg2, deg_raw, b2_2):
    return pl.pallas_call(
        _epi_body,
        grid=(GRID,),
        in_specs=[
            pl.BlockSpec((2, RT, DH), lambda i: (0, i, 0)),
            pl.BlockSpec((2, RT, 16), lambda i: (0, i, 0)),
            pl.BlockSpec((1, 128), lambda i: (0, 0)),
        ],
        out_specs=pl.BlockSpec((RT, 128), lambda i: (i, 0)),
        out_shape=jax.ShapeDtypeStruct((N, 128), jnp.float32),
    )(agg2, deg_raw, b2_2)


def kernel(x, edge_index, W1, b1, W2, b2):
    src = edge_index[0]
    dst = edge_index[1]
    srcr = jnp.stack([src, src + NPAD]).reshape(1, NC, NS, NCHUNKS, CHUNK)
    dstr = dst.reshape(NS, NCHUNKS, CHUNK)
    dstr32 = dst.reshape(NC * NS, DEG_NCHUNKS, CHUNK)

    deg_raw = _build_deg_kernel()(dstr32)
    hs1 = _mm1(x, W1, deg_raw).reshape(2, 2 * NPAD, DH)
    agg1a = _build_spmm(1)(srcr, dstr, hs1[0])
    agg1b = _build_spmm(1)(srcr, dstr, hs1[1])
    hs2 = _mm2(agg1a, agg1b, deg_raw, b1.reshape(2, 128), W2.reshape(2, 128, 128))
    agg2 = _build_spmm(1)(srcr, dstr, hs2.reshape(2 * NPAD, DH))
    return _epi(agg2, deg_raw, b2.reshape(1, 128))
